# baseline (speedup 1.0000x reference)
import functools

import jax
import jax.numpy as jnp
from jax import lax
from jax.experimental import pallas as pl
from jax.experimental.pallas import tpu as pltpu
from jax.experimental.pallas import tpu_sc as plsc

VOCAB = 100000
EMB = 64
BATCH = 4096
CTX = 50

NC = 2
NS = 16
NW = NC * NS
BPW = BATCH // NW
CB = 16
CHUNKS = BPW // CB
ROWS = CB * CTX
GROWS = 100
GSUB = ROWS // GROWS
LANES = 16
EL = EMB // LANES

NVB = 1024


def _pool_sc(xr, emb):
    mesh = plsc.VectorSubcoreMesh(core_axis_name="c", subcore_axis_name="s")

    @functools.partial(
        pl.kernel,
        mesh=mesh,
        compiler_params=pltpu.CompilerParams(use_tc_tiling_on_sc=False),
        out_type=jax.ShapeDtypeStruct((BATCH, EMB), jnp.float32),
        scratch_types=[
            pltpu.VMEM((GSUB, GROWS), jnp.int32),
            pltpu.VMEM((ROWS, EMB), jnp.float32),
            pltpu.VMEM((CB, EMB), jnp.float32),
            pltpu.SemaphoreType.DMA,
        ],
    )
    def pool(xr_hbm, emb_hbm, out_hbm, idx_v, rows_v, acc_v, sem):
        wid = lax.axis_index("s") * NC + lax.axis_index("c")
        for ci in range(CHUNKS):
            base = wid * BPW + ci * CB
            irow = wid * (BPW * CTX // GROWS) + ci * (CB * CTX // GROWS)
            pltpu.sync_copy(xr_hbm.at[pl.ds(irow, GSUB)], idx_v)
            copies = [
                pltpu.async_copy(
                    emb_hbm.at[idx_v.at[j]],
                    rows_v.at[pl.ds(j * GROWS, GROWS)],
                    sem,
                )
                for j in range(GSUB)
            ]
            for cp in copies:
                cp.wait()

            def bbody(b, _):
                r0 = b * CTX

                def cbody(c, accs):
                    return tuple(
                        accs[l] + rows_v[r0 + c, pl.ds(l * LANES, LANES)]
                        for l in range(EL)
                    )

                accs = lax.fori_loop(
                    0, CTX, cbody,
                    tuple(jnp.zeros((LANES,), jnp.float32) for _ in range(EL)),
                )
                for l in range(EL):
                    acc_v[b, pl.ds(l * LANES, LANES)] = accs[l] * (1.0 / CTX)
                return 0

            lax.fori_loop(0, CB, bbody, 0)
            pltpu.sync_copy(acc_v, out_hbm.at[pl.ds(base, CB)])

    return pool(xr, emb)


def _project_tc(pooled, W):
    def mm(x_ref, w_ref, o_ref):
        o_ref[...] = lax.dot_general(
            x_ref[...], w_ref[...],
            dimension_numbers=(((1,), (1,)), ((), ())),
            preferred_element_type=jnp.float32,
        )

    return pl.pallas_call(
        mm,
        grid=(pl.cdiv(VOCAB, NVB),),
        in_specs=[
            pl.BlockSpec((BATCH, EMB), lambda j: (0, 0)),
            pl.BlockSpec((NVB, EMB), lambda j: (j, 0)),
        ],
        out_specs=pl.BlockSpec((BATCH, NVB), lambda j: (0, j)),
        out_shape=jax.ShapeDtypeStruct((BATCH, VOCAB), jnp.float32),
    )(pooled, W)


def kernel(x, emb, W):
    xr = x.reshape(BATCH * CTX // GROWS, GROWS)
    pooled = _pool_sc(xr, emb)
    return _project_tc(pooled, W)
